# Initial kernel scaffold; baseline (speedup 1.0000x reference)
import jax
import jax.numpy as jnp
from jax.experimental import pallas as pl


def kernel(x, edge_index, batch, W1, b1, w_rel, b_rel, w_root, W2, b2):
    raise NotImplementedError("write your pallas kernel here")


# speedup vs baseline: 2.8027x; 2.8027x over previous
import functools

import jax
import jax.numpy as jnp
from jax import lax
from jax.experimental import pallas as pl
from jax.experimental.pallas import tpu as pltpu
from jax.experimental.pallas import tpu_sc as plsc

N = 10000
E = 320000
D = 128
K = 5000
NP = 10016
KP = 5120
KO = 5008
NW = 32
ECH = 10000
RB = 192
MCAP = 12288
MCAP2 = 8128

_f32 = jnp.float32
_i32 = jnp.int32


def _wrange(wid):
    n0 = wid * 312 + jnp.minimum(wid, 2) * 8
    cnt = jnp.where(wid < 2, 320, 312)
    return n0, cnt


def _mesh():
    return plsc.VectorSubcoreMesh(core_axis_name="c", subcore_axis_name="s")


def _wid():
    return lax.axis_index("s") * 2 + lax.axis_index("c")


def _prefix16(v):
    io = lax.iota(_i32, 16)
    acc = jnp.zeros((16,), _i32)
    for l in range(16):
        acc = acc + jnp.where(io >= l, v[l], 0)
    return acc


def _sget(ref, i):
    return ref[pl.ds(i, 16)][0]


def _sset(ref, i, val):
    v = ref[pl.ds(i, 16)]
    ref[pl.ds(i, 16)] = jnp.where(lax.iota(_i32, 16) == 0, val, v)


def _deg_partials(dst):
    @functools.partial(
        pl.kernel,
        out_type=jax.ShapeDtypeStruct((NW, 1, NP), _i32),
        mesh=_mesh(),
        scratch_types=[
            pltpu.VMEM((ECH,), _i32),
            pltpu.VMEM((NP,), _i32),
        ],
    )
    def k(dst_hbm, out_hbm, ebuf, hist):
        wid = _wid()

        def zero(i, _):
            hist[pl.ds(i * 16, 16)] = jnp.zeros((16,), _i32)
            return 0

        lax.fori_loop(0, NP // 16, zero, 0)
        pltpu.sync_copy(dst_hbm.at[pl.ds(wid * (E // NW), ECH)], ebuf)

        def upd(i, _):
            dv = ebuf[pl.ds(i * 16, 16)]
            for l in range(16):
                d = dv[l]
                _sset(hist, d, _sget(hist, d) + 1)
            return 0

        lax.fori_loop(0, ECH // 16, upd, 0)
        pltpu.sync_copy(hist, out_hbm.at[wid, 0])

    return k(dst)


def _t1(x, W1, part):
    def body(x_ref, w_ref, p_ref, xw_ref, dis_ref):
        xw = jnp.dot(x_ref[...], w_ref[...], preferred_element_type=_f32)
        xw_ref[pl.ds(0, N), :] = xw
        xw_ref[pl.ds(N, NP - N), :] = jnp.zeros((NP - N, D), _f32)
        deg = jnp.sum(p_ref[...], axis=0) + 1
        dis_ref[...] = lax.rsqrt(deg.astype(_f32))

    return pl.pallas_call(
        body,
        out_shape=[
            jax.ShapeDtypeStruct((NP, D), _f32),
            jax.ShapeDtypeStruct((1, NP), _f32),
        ],
    )(x, W1, part)


def _seg_sum(mode, table, src, dst, dis=None, bias=None):
    conv1 = mode == "conv1"
    scratch = [
        pltpu.VMEM((ECH,), _i32),
        pltpu.VMEM((ECH,), _i32),
        pltpu.VMEM((MCAP + 16,), _i32),
        pltpu.VMEM((MCAP + 16,), _i32),
        pltpu.VMEM((320, D), _f32),
        pltpu.VMEM((RB, D), _f32),
        pltpu.SemaphoreType.DMA,
    ]
    if conv1:
        scratch += [
            pltpu.VMEM((NP,), _f32),
            pltpu.VMEM((16, D), _f32),
            pltpu.VMEM((D,), _f32),
        ]

    @functools.partial(
        pl.kernel,
        out_type=jax.ShapeDtypeStruct((N, D), _f32),
        mesh=_mesh(),
        scratch_types=scratch,
    )
    def k(table_hbm, src_hbm, dst_hbm, *rest):
        if conv1:
            (dis_hbm, b_hbm, out_hbm, dbuf, sbuf, msrc, mdst, acc, rows,
             sem, disv, selfb, bv) = rest
        else:
            (out_hbm, dbuf, sbuf, msrc, mdst, acc, rows, sem) = rest
        wid = _wid()
        n0, cnt = _wrange(wid)
        if conv1:
            pltpu.sync_copy(dis_hbm, disv)
            pltpu.sync_copy(b_hbm, bv)

        def zrow(i, _):
            for c in range(8):
                acc[i, pl.ds(c * 16, 16)] = jnp.zeros((16,), _f32)
            return 0

        lax.fori_loop(0, 320, zrow, 0)

        def zidx(i, _):
            msrc[pl.ds(i * 16, 16)] = jnp.zeros((16,), _i32)
            return 0

        lax.fori_loop(0, (MCAP + 16) // 16, zidx, 0)

        def chunk(ch, off):
            pltpu.sync_copy(dst_hbm.at[pl.ds(ch * ECH, ECH)], dbuf)
            pltpu.sync_copy(src_hbm.at[pl.ds(ch * ECH, ECH)], sbuf)

            def inner(j, off):
                vd = dbuf[pl.ds(j * 16, 16)]
                vs = sbuf[pl.ds(j * 16, 16)]
                m = (vd >= n0) & (vd < n0 + cnt)
                mv = jnp.where(m, 1, 0).astype(_i32)
                nm = mv[0]
                for l in range(1, 16):
                    nm = nm + mv[l]
                anym = nm > 0

                def wr(off):
                    for l in range(16):
                        _sset(mdst, off, vd[l])
                        _sset(msrc, off, vs[l])
                        off = off + mv[l]
                    return off

                off = lax.cond(anym, wr, lambda off: off, off)
                return jnp.minimum(off, MCAP)

            return lax.fori_loop(0, ECH // 16, inner, off)

        off = lax.fori_loop(0, E // ECH, chunk, jnp.int32(0))

        nb = (off + RB - 1) // RB

        def blk(b, _):
            base = b * RB
            pltpu.async_copy(
                table_hbm.at[msrc.at[pl.ds(base, RB)]], rows, sem
            ).wait()

            def egrp(g, _):
                gb = base + g * 16
                vs = msrc[pl.ds(gb, 16)]
                vd = mdst[pl.ds(gb, 16)]
                for l in range(16):
                    e = gb + l

                    @pl.when(e < off)
                    def _(l=l):
                        d = vd[l]
                        ln = d - n0
                        i = g * 16 + l
                        if conv1:
                            nrm = _sget(disv, vs[l]) * _sget(disv, d)
                        for c in range(8):
                            sl = pl.ds(c * 16, 16)
                            r = rows[i, sl]
                            if conv1:
                                r = r * nrm
                            acc[ln, sl] = acc[ln, sl] + r
                return 0

            lax.fori_loop(0, RB // 16, egrp, 0)
            return 0

        lax.fori_loop(0, nb, blk, 0)

        if conv1:
            def grp(g, _):
                pltpu.sync_copy(
                    table_hbm.at[pl.ds(n0 + g * 16, 16)], selfb
                )
                for t in range(16):
                    ln = g * 16 + t

                    @pl.when(ln < cnt)
                    def _(t=t, ln=ln):
                        dd = _sget(disv, jnp.minimum(n0 + ln, N - 1))
                        sn = dd * dd
                        for c in range(8):
                            sl = pl.ds(c * 16, 16)
                            v = acc[ln, sl] + selfb[t, sl] * sn
                            v = v + bv[sl]
                            acc[ln, sl] = jnp.where(v > 0.0, v, 0.0)
                return 0

            lax.fori_loop(0, 20, grp, 0)

        @pl.when(cnt == 320)
        def _():
            pltpu.sync_copy(acc, out_hbm.at[pl.ds(n0, 320)])

        @pl.when(cnt == 312)
        def _():
            pltpu.sync_copy(acc.at[pl.ds(0, 312)], out_hbm.at[pl.ds(n0, 312)])

    if conv1:
        return k(table, src, dst, dis, bias)
    return k(table, src, dst)


def _t2(aggr, h, w_rel, b_rel, w_root):
    def body(a_ref, h_ref, wr_ref, br_ref, wo_ref, s_ref):
        za = jnp.dot(a_ref[...], wr_ref[...], preferred_element_type=_f32)
        h16 = h_ref[...].astype(jnp.bfloat16)
        zr = jnp.dot(h16, wo_ref[...], preferred_element_type=_f32)
        z = (za + br_ref[...]) + zr
        s_ref[...] = jnp.tanh(z)

    return pl.pallas_call(
        body,
        out_shape=jax.ShapeDtypeStruct((N, 1), _f32),
    )(aggr, h, w_rel, b_rel, w_root)


def _topk(score):
    NB = 2048

    @functools.partial(
        pl.kernel,
        out_type=[
            jax.ShapeDtypeStruct((KP,), _i32),
            jax.ShapeDtypeStruct((NP,), _i32),
            jax.ShapeDtypeStruct((NP,), _i32),
            jax.ShapeDtypeStruct((KP,), _f32),
        ],
        mesh=_mesh(),
        scratch_types=[
            pltpu.VMEM((N + 16,), _f32),
            pltpu.VMEM((N + 16,), _i32),
            pltpu.VMEM((N + 16,), _i32),
            pltpu.VMEM((N + 16,), _i32),
            pltpu.VMEM((N + 16,), _i32),
            pltpu.VMEM((NB + 16,), _i32),
            pltpu.VMEM((N + 16,), _i32),
            pltpu.VMEM((NP,), _i32),
            pltpu.VMEM((NP,), _i32),
        ],
    )
    def k(score_hbm, perm_hbm, sel_hbm, nidx_hbm, ssort_hbm,
          sbuf, k0, p0, k1, p1, bins, dig, selb, nidxb):
        wid = _wid()

        @pl.when(wid == 0)
        def _():
            pltpu.sync_copy(score_hbm, sbuf.at[pl.ds(0, N)])

            def mk(j, _):
                v = sbuf[pl.ds(j * 16, 16)]
                b = lax.bitcast_convert_type(v, _i32)
                b = jnp.where(b == jnp.int32(-2147483648), 0, b)
                m = jnp.where(b < 0, ~b, b ^ jnp.int32(-2147483648))
                u = ~m
                k0[pl.ds(j * 16, 16)] = u
                p0[pl.ds(j * 16, 16)] = lax.iota(_i32, 16) + j * 16
                return 0

            lax.fori_loop(0, N // 16, mk, 0)

            for pno, shift in enumerate([0, 11, 22]):
                ks, ps = (k0, p0) if pno % 2 == 0 else (k1, p1)
                kd, pd = (k1, p1) if pno % 2 == 0 else (k0, p0)

                def zb(i, _):
                    bins[pl.ds(i * 16, 16)] = jnp.zeros((16,), _i32)
                    return 0

                lax.fori_loop(0, NB // 16, zb, 0)

                def dg(j, _, ks=ks, shift=shift):
                    u = ks[pl.ds(j * 16, 16)]
                    d = lax.shift_right_logical(u, shift) & (NB - 1)
                    dig[pl.ds(j * 16, 16)] = d
                    return 0

                lax.fori_loop(0, N // 16, dg, 0)

                def hist(i, _):
                    dv = dig[pl.ds(i * 16, 16)]
                    for l in range(16):
                        d = dv[l]
                        _sset(bins, d, _sget(bins, d) + 1)
                    return 0

                lax.fori_loop(0, N // 16, hist, 0)

                def scan(i, carry):
                    v = bins[pl.ds(i * 16, 16)]
                    inc = _prefix16(v)
                    exc = inc - v + carry
                    bins[pl.ds(i * 16, 16)] = exc
                    return carry + inc[15]

                lax.fori_loop(0, NB // 16, scan, jnp.int32(0))

                def perm_(i, _, ks=ks, ps=ps, kd=kd, pd=pd):
                    dv = dig[pl.ds(i * 16, 16)]
                    kv = ks[pl.ds(i * 16, 16)]
                    pv = ps[pl.ds(i * 16, 16)]
                    for l in range(16):
                        d = dv[l]
                        slot = _sget(bins, d)
                        _sset(bins, d, slot + 1)
                        _sset(kd, slot, kv[l])
                        _sset(pd, slot, pv[l])
                    return 0

                lax.fori_loop(0, N // 16, perm_, 0)

            pltpu.sync_copy(p1.at[pl.ds(0, KP)], perm_hbm)

            def inv(j, _):
                u = k1[pl.ds(j * 16, 16)]
                m = ~u
                b = jnp.where(m < 0, m ^ jnp.int32(-2147483648), ~m)
                sbuf[pl.ds(j * 16, 16)] = lax.bitcast_convert_type(b, _f32)
                return 0

            lax.fori_loop(0, KP // 16, inv, 0)
            pltpu.sync_copy(sbuf.at[pl.ds(0, KP)], ssort_hbm)

            def zsel(i, _):
                selb[pl.ds(i * 16, 16)] = jnp.zeros((16,), _i32)
                nidxb[pl.ds(i * 16, 16)] = jnp.zeros((16,), _i32)
                return 0

            lax.fori_loop(0, NP // 16, zsel, 0)

            def mark(r, _):
                pv = p1[pl.ds(r * 16, 16)]
                for l in range(16):
                    e = r * 16 + l

                    @pl.when(e < K)
                    def _():
                        i = pv[l]
                        _sset(selb, i, 1)
                        _sset(nidxb, i, e)
                return 0

            lax.fori_loop(0, (K + 15) // 16, mark, 0)
            pltpu.sync_copy(selb, sel_hbm)
            pltpu.sync_copy(nidxb, nidx_hbm)

    return k(score)


def _hp_deg2(h, perm, ssort, src, dst, sel, nidx):
    @functools.partial(
        pl.kernel,
        out_type=[
            jax.ShapeDtypeStruct((KP, D), _f32),
            jax.ShapeDtypeStruct((NW, 1, KO), _i32),
        ],
        mesh=_mesh(),
        scratch_types=[
            pltpu.VMEM((160,), _i32),
            pltpu.VMEM((160,), _f32),
            pltpu.VMEM((160, D), _f32),
            pltpu.VMEM((ECH,), _i32),
            pltpu.VMEM((ECH,), _i32),
            pltpu.VMEM((NP,), _i32),
            pltpu.VMEM((NP,), _i32),
            pltpu.VMEM((KO + 16,), _i32),
            pltpu.SemaphoreType.DMA,
        ],
    )
    def k(h_hbm, perm_hbm, ss_hbm, src_hbm, dst_hbm, sel_hbm, nidx_hbm,
          hp_hbm, d2_hbm, pbuf, scbuf, rows, dbuf, sbuf, selv, nidxv,
          hist, sem):
        wid = _wid()
        base = wid * 160
        pltpu.sync_copy(perm_hbm.at[pl.ds(base, 160)], pbuf)
        pltpu.sync_copy(ss_hbm.at[pl.ds(base, 160)], scbuf)
        pltpu.async_copy(h_hbm.at[pbuf], rows, sem).wait()

        def mul(g, _):
            sv = scbuf[pl.ds(g * 16, 16)]
            for l in range(16):
                i = g * 16 + l
                for c in range(8):
                    sl = pl.ds(c * 16, 16)
                    rows[i, sl] = rows[i, sl] * sv[l]
            return 0

        lax.fori_loop(0, 10, mul, 0)
        pltpu.sync_copy(rows, hp_hbm.at[pl.ds(base, 160)])

        pltpu.sync_copy(sel_hbm, selv)
        pltpu.sync_copy(nidx_hbm, nidxv)

        def zh(i, _):
            hist[pl.ds(i * 16, 16)] = jnp.zeros((16,), _i32)
            return 0

        lax.fori_loop(0, (KO + 16) // 16, zh, 0)
        pltpu.sync_copy(dst_hbm.at[pl.ds(wid * (E // NW), ECH)], dbuf)
        pltpu.sync_copy(src_hbm.at[pl.ds(wid * (E // NW), ECH)], sbuf)

        def upd(i, _):
            vs = sbuf[pl.ds(i * 16, 16)]
            vd = dbuf[pl.ds(i * 16, 16)]
            for l in range(16):
                s = vs[l]
                d = vd[l]
                ok = (_sget(selv, s) == 1) & (_sget(selv, d) == 1)

                @pl.when(ok)
                def _(d=d):
                    j = _sget(nidxv, d)
                    _sset(hist, j, _sget(hist, j) + 1)
            return 0

        lax.fori_loop(0, ECH // 16, upd, 0)
        pltpu.sync_copy(hist.at[pl.ds(0, KO)], d2_hbm.at[wid, 0])

    return k(h, perm, ssort, src, dst, sel, nidx)


def _t3(hp, W2, part2):
    def body(hp_ref, w_ref, p_ref, xw_ref, dis_ref):
        xw_ref[...] = jnp.dot(
            hp_ref[...], w_ref[...], preferred_element_type=_f32
        )
        deg = jnp.sum(p_ref[...], axis=0) + 1
        dis_ref[...] = lax.rsqrt(deg.astype(_f32))

    return pl.pallas_call(
        body,
        out_shape=[
            jax.ShapeDtypeStruct((KP, D), _f32),
            jax.ShapeDtypeStruct((1, KO), _f32),
        ],
    )(hp, W2, part2)


def _conv2(xw2, src, dst, sel, nidx, dis2, b2):
    @functools.partial(
        pl.kernel,
        out_type=jax.ShapeDtypeStruct((KO, D), _f32),
        mesh=_mesh(),
        scratch_types=[
            pltpu.VMEM((ECH,), _i32),
            pltpu.VMEM((ECH,), _i32),
            pltpu.VMEM((MCAP2 + 16,), _i32),
            pltpu.VMEM((MCAP2 + 16,), _i32),
            pltpu.VMEM((320, D), _f32),
            pltpu.VMEM((RB, D), _f32),
            pltpu.VMEM((NP,), _i32),
            pltpu.VMEM((NP,), _i32),
            pltpu.VMEM((KO + 16,), _f32),
            pltpu.VMEM((D,), _f32),
            pltpu.VMEM((16, D), _f32),
            pltpu.VMEM((16,), _i32),
            pltpu.SemaphoreType.DMA,
        ],
    )
    def k(xw_hbm, src_hbm, dst_hbm, sel_hbm, nidx_hbm, dis_hbm, b_hbm,
          out_hbm, dbuf, sbuf, msrc, mdst, acc, rows, selv, nidxv, disv,
          bv, selfb, jidx, sem):
        wid = _wid()
        n0, cnt = _wrange(wid)
        pltpu.sync_copy(sel_hbm, selv)
        pltpu.sync_copy(nidx_hbm, nidxv)
        pltpu.sync_copy(dis_hbm, disv.at[pl.ds(0, KO)])
        pltpu.sync_copy(b_hbm, bv)

        def zrow(i, _):
            for c in range(8):
                acc[i, pl.ds(c * 16, 16)] = jnp.zeros((16,), _f32)
            return 0

        lax.fori_loop(0, 320, zrow, 0)

        def zidx(i, _):
            msrc[pl.ds(i * 16, 16)] = jnp.zeros((16,), _i32)
            return 0

        lax.fori_loop(0, (MCAP2 + 16) // 16, zidx, 0)

        def chunk(ch, off):
            pltpu.sync_copy(dst_hbm.at[pl.ds(ch * ECH, ECH)], dbuf)
            pltpu.sync_copy(src_hbm.at[pl.ds(ch * ECH, ECH)], sbuf)

            def inner(j, off):
                vd = dbuf[pl.ds(j * 16, 16)]
                vs = sbuf[pl.ds(j * 16, 16)]
                m = (vd >= n0) & (vd < n0 + cnt)
                mv = jnp.where(m, 1, 0).astype(_i32)
                nm = mv[0]
                for l in range(1, 16):
                    nm = nm + mv[l]

                def wr(off):
                    for l in range(16):
                        s = vs[l]
                        d = vd[l]
                        ok = ((mv[l] == 1) & (_sget(selv, s) == 1)
                              & (_sget(selv, d) == 1))

                        @pl.when(ok)
                        def _(s=s, d=d, off=off):
                            _sset(mdst, off, d)
                            _sset(msrc, off, _sget(nidxv, s))
                        off = off + jnp.where(ok, 1, 0)
                    return off

                off = lax.cond(nm > 0, wr, lambda off: off, off)
                return jnp.minimum(off, MCAP2)

            return lax.fori_loop(0, ECH // 16, inner, off)

        off = lax.fori_loop(0, E // ECH, chunk, jnp.int32(0))

        nb = (off + RB - 1) // RB

        def blk(b, _):
            base = b * RB
            pltpu.async_copy(
                xw_hbm.at[msrc.at[pl.ds(base, RB)]], rows, sem
            ).wait()

            def egrp(g, _):
                gb = base + g * 16
                ns = msrc[pl.ds(gb, 16)]
                vd = mdst[pl.ds(gb, 16)]
                for l in range(16):
                    e = gb + l

                    @pl.when(e < off)
                    def _(l=l):
                        d = vd[l]
                        ln = d - n0
                        i = g * 16 + l
                        jj = _sget(nidxv, d)
                        nrm = _sget(disv, ns[l]) * _sget(disv, jj)
                        for c in range(8):
                            sl = pl.ds(c * 16, 16)
                            acc[ln, sl] = acc[ln, sl] + rows[i, sl] * nrm
                return 0

            lax.fori_loop(0, RB // 16, egrp, 0)
            return 0

        lax.fori_loop(0, nb, blk, 0)

        def grp(g, _):
            base_old = n0 + g * 16
            io16 = lax.iota(_i32, 16)
            jv = jnp.full((16,), KO - 1, _i32)
            okv = jnp.zeros((16,), _i32)
            for t in range(16):
                d = jnp.minimum(base_old + t, N - 1)
                okt = (g * 16 + t < cnt) & (_sget(selv, d) == 1)
                jt = jnp.where(okt, _sget(nidxv, d), KO - 1)
                jv = jnp.where(io16 == t, jt, jv)
                okv = jnp.where(io16 == t, jnp.where(okt, 1, 0), okv)
            jidx[...] = jv
            pltpu.async_copy(xw_hbm.at[jidx], selfb, sem).wait()

            for t in range(16):
                ln = g * 16 + t
                ok = okv[t] == 1

                @pl.when(ok)
                def _(t=t, ln=ln):
                    dj = _sget(disv, jv[t])
                    sn = dj * dj
                    for c in range(8):
                        sl = pl.ds(c * 16, 16)
                        v = acc[ln, sl] + selfb[t, sl] * sn
                        v = v + bv[sl]
                        selfb[t, sl] = jnp.where(v > 0.0, v, 0.0)

                @pl.when(jnp.logical_not(ok))
                def _(t=t):
                    for c in range(8):
                        selfb[t, pl.ds(c * 16, 16)] = jnp.zeros((16,), _f32)
            pltpu.async_copy(selfb, out_hbm.at[jidx], sem).wait()
            return 0

        lax.fori_loop(0, 20, grp, 0)

    return k(xw2, src, dst, sel, nidx, dis2, b2)


def kernel(x, edge_index, batch, W1, b1, w_rel, b_rel, w_root, W2, b2):
    del batch
    src = edge_index[0]
    dst = edge_index[1]
    part = _deg_partials(dst)
    xw, dis2d = _t1(x, W1, part)
    dis = dis2d.reshape(NP)
    h = _seg_sum("conv1", xw, src, dst, dis=dis, bias=b1)
    aggr = _seg_sum("aggr", h, src, dst)
    score = _t2(aggr, h, w_rel, b_rel, w_root).reshape(N)
    perm, sel, nidx, ssort = _topk(score)
    hp, part2 = _hp_deg2(h, perm, ssort, src, dst, sel, nidx)
    xw2, dis2_2d = _t3(hp, W2, part2)
    dis2 = dis2_2d.reshape(KO)
    out = _conv2(xw2, src, dst, sel, nidx, dis2, b2)
    return out[:K]


# speedup vs baseline: 3.2183x; 1.1483x over previous
import functools

import jax
import jax.numpy as jnp
from jax import lax
from jax.experimental import pallas as pl
from jax.experimental.pallas import tpu as pltpu
from jax.experimental.pallas import tpu_sc as plsc

N = 10000
E = 320000
D = 128
K = 5000
NP = 10016
KP = 5120
KO = 5008
NW = 32
ECH = 10000
RB = 192
MCAP = 12288
MCAP2 = 8128

_f32 = jnp.float32
_i32 = jnp.int32


def _wrange(wid):
    n0 = wid * 312 + jnp.minimum(wid, 2) * 8
    cnt = jnp.where(wid < 2, 320, 312)
    return n0, cnt


def _mesh():
    return plsc.VectorSubcoreMesh(core_axis_name="c", subcore_axis_name="s")


def _wid():
    return lax.axis_index("s") * 2 + lax.axis_index("c")


def _prefix16(v):
    io = lax.iota(_i32, 16)
    acc = jnp.zeros((16,), _i32)
    for l in range(16):
        acc = acc + jnp.where(io >= l, v[l], 0)
    return acc


def _sget(ref, i):
    return ref[pl.ds(i, 16)][0]


def _sset(ref, i, val):
    v = ref[pl.ds(i, 16)]
    ref[pl.ds(i, 16)] = jnp.where(lax.iota(_i32, 16) == 0, val, v)


def _deg_partials(dst):
    @functools.partial(
        pl.kernel,
        out_type=jax.ShapeDtypeStruct((NW, 1, NP), _i32),
        mesh=_mesh(),
        scratch_types=[
            pltpu.VMEM((ECH,), _i32),
            pltpu.VMEM((NP,), _i32),
        ],
    )
    def k(dst_hbm, out_hbm, ebuf, hist):
        wid = _wid()

        def zero(i, _):
            hist[pl.ds(i * 16, 16)] = jnp.zeros((16,), _i32)
            return 0

        lax.fori_loop(0, NP // 16, zero, 0)
        pltpu.sync_copy(dst_hbm.at[pl.ds(wid * (E // NW), ECH)], ebuf)

        def upd(i, _):
            dv = ebuf[pl.ds(i * 16, 16)]
            for l in range(16):
                d = dv[l]
                _sset(hist, d, _sget(hist, d) + 1)
            return 0

        lax.fori_loop(0, ECH // 16, upd, 0)
        pltpu.sync_copy(hist, out_hbm.at[wid, 0])

    return k(dst)


def _t1(x, W1, part):
    def body(x_ref, w_ref, p_ref, xw_ref, dis_ref):
        xw = jnp.dot(x_ref[...], w_ref[...], preferred_element_type=_f32)
        xw_ref[pl.ds(0, N), :] = xw
        xw_ref[pl.ds(N, NP - N), :] = jnp.zeros((NP - N, D), _f32)
        deg = jnp.sum(p_ref[...], axis=0) + 1
        dis_ref[...] = lax.rsqrt(deg.astype(_f32))

    return pl.pallas_call(
        body,
        out_shape=[
            jax.ShapeDtypeStruct((NP, D), _f32),
            jax.ShapeDtypeStruct((1, NP), _f32),
        ],
    )(x, W1, part)


def _seg_sum(mode, table, src, dst, dis=None, bias=None, offs=None):
    conv1 = mode == "conv1"
    scratch = [
        pltpu.VMEM((ECH,), _i32),
        pltpu.VMEM((ECH,), _i32),
        pltpu.VMEM((MCAP + 16,), _i32),
        pltpu.VMEM((MCAP + 16,), _i32),
        pltpu.VMEM((320, D), _f32),
        pltpu.VMEM((RB, D), _f32),
        pltpu.SemaphoreType.DMA,
    ]
    if conv1:
        scratch += [
            pltpu.VMEM((NP,), _f32),
            pltpu.VMEM((16, D), _f32),
            pltpu.VMEM((D,), _f32),
        ]

    if conv1:
        outs = [
            jax.ShapeDtypeStruct((N, D), _f32),
            jax.ShapeDtypeStruct((NW, 1, MCAP + 16), _i32),
            jax.ShapeDtypeStruct((NW, 1, MCAP + 16), _i32),
            jax.ShapeDtypeStruct((NW, 1, 16), _i32),
        ]
        scratch.append(pltpu.VMEM((16,), _i32))
    else:
        outs = jax.ShapeDtypeStruct((N, D), _f32)

    @functools.partial(
        pl.kernel,
        out_type=outs,
        mesh=_mesh(),
        scratch_types=scratch,
    )
    def k(table_hbm, src_hbm, dst_hbm, *rest):
        if conv1:
            (dis_hbm, b_hbm, out_hbm, ms_hbm, md_hbm, off_hbm,
             dbuf, sbuf, msrc, mdst, acc, rows,
             sem, disv, selfb, bv, offbuf) = rest
        else:
            (offs_hbm, out_hbm, dbuf, sbuf, msrc, mdst, acc, rows, sem) = rest
        wid = _wid()
        n0, cnt = _wrange(wid)
        if conv1:
            pltpu.sync_copy(dis_hbm, disv)
            pltpu.sync_copy(b_hbm, bv)

        def zrow(i, _):
            for c in range(8):
                acc[i, pl.ds(c * 16, 16)] = jnp.zeros((16,), _f32)
            return 0

        lax.fori_loop(0, 320, zrow, 0)

        if conv1:
            def zidx(i, _):
                msrc[pl.ds(i * 16, 16)] = jnp.zeros((16,), _i32)
                return 0

            lax.fori_loop(0, (MCAP + 16) // 16, zidx, 0)

            def chunk(ch, off):
                pltpu.sync_copy(dst_hbm.at[pl.ds(ch * ECH, ECH)], dbuf)
                pltpu.sync_copy(src_hbm.at[pl.ds(ch * ECH, ECH)], sbuf)

                def inner(j, off):
                    vd = dbuf[pl.ds(j * 16, 16)]
                    vs = sbuf[pl.ds(j * 16, 16)]
                    m = (vd >= n0) & (vd < n0 + cnt)
                    mv = jnp.where(m, 1, 0).astype(_i32)
                    nm = mv[0]
                    for l in range(1, 16):
                        nm = nm + mv[l]
                    anym = nm > 0

                    def wr(off):
                        for l in range(16):
                            _sset(mdst, off, vd[l])
                            _sset(msrc, off, vs[l])
                            off = off + mv[l]
                        return off

                    off = lax.cond(anym, wr, lambda off: off, off)
                    return jnp.minimum(off, MCAP)

                return lax.fori_loop(0, ECH // 16, inner, off)

            off = lax.fori_loop(0, E // ECH, chunk, jnp.int32(0))
            offbuf[...] = jnp.where(
                lax.iota(_i32, 16) == 0, off, 0
            )
            pltpu.sync_copy(offbuf, off_hbm.at[wid, 0])
            pltpu.sync_copy(msrc, ms_hbm.at[wid, 0])
            pltpu.sync_copy(mdst, md_hbm.at[wid, 0])
        else:
            pltpu.sync_copy(src_hbm.at[wid, 0], msrc)
            pltpu.sync_copy(dst_hbm.at[wid, 0], mdst)
            pltpu.sync_copy(offs_hbm.at[wid, 0], dbuf.at[pl.ds(0, 16)])
            off = dbuf[pl.ds(0, 16)][0]

        nb = (off + RB - 1) // RB

        def blk(b, _):
            base = b * RB
            pltpu.async_copy(
                table_hbm.at[msrc.at[pl.ds(base, RB)]], rows, sem
            ).wait()

            def egrp(g, _):
                gb = base + g * 16
                vs = msrc[pl.ds(gb, 16)]
                vd = mdst[pl.ds(gb, 16)]
                for l in range(16):
                    e = gb + l

                    @pl.when(e < off)
                    def _(l=l):
                        d = vd[l]
                        ln = d - n0
                        i = g * 16 + l
                        if conv1:
                            nrm = _sget(disv, vs[l]) * _sget(disv, d)
                        for c in range(8):
                            sl = pl.ds(c * 16, 16)
                            r = rows[i, sl]
                            if conv1:
                                r = r * nrm
                            acc[ln, sl] = acc[ln, sl] + r
                return 0

            lax.fori_loop(0, RB // 16, egrp, 0)
            return 0

        lax.fori_loop(0, nb, blk, 0)

        if conv1:
            def grp(g, _):
                pltpu.sync_copy(
                    table_hbm.at[pl.ds(n0 + g * 16, 16)], selfb
                )
                for t in range(16):
                    ln = g * 16 + t

                    @pl.when(ln < cnt)
                    def _(t=t, ln=ln):
                        dd = _sget(disv, jnp.minimum(n0 + ln, N - 1))
                        sn = dd * dd
                        for c in range(8):
                            sl = pl.ds(c * 16, 16)
                            v = acc[ln, sl] + selfb[t, sl] * sn
                            v = v + bv[sl]
                            acc[ln, sl] = jnp.where(v > 0.0, v, 0.0)
                return 0

            lax.fori_loop(0, 20, grp, 0)

        @pl.when(cnt == 320)
        def _():
            pltpu.sync_copy(acc, out_hbm.at[pl.ds(n0, 320)])

        @pl.when(cnt == 312)
        def _():
            pltpu.sync_copy(acc.at[pl.ds(0, 312)], out_hbm.at[pl.ds(n0, 312)])

    if conv1:
        return k(table, src, dst, dis, bias)
    return k(table, src, dst, offs)


def _t2(aggr, h, w_rel, b_rel, w_root):
    def body(a_ref, h_ref, wr_ref, br_ref, wo_ref, s_ref):
        za = jnp.dot(a_ref[...], wr_ref[...], preferred_element_type=_f32)
        h16 = h_ref[...].astype(jnp.bfloat16)
        zr = jnp.dot(h16, wo_ref[...], preferred_element_type=_f32)
        z = (za + br_ref[...]) + zr
        s_ref[...] = jnp.tanh(z)

    return pl.pallas_call(
        body,
        out_shape=jax.ShapeDtypeStruct((N, 1), _f32),
    )(aggr, h, w_rel, b_rel, w_root)


def _topk(score):
    NB = 2048

    @functools.partial(
        pl.kernel,
        out_type=[
            jax.ShapeDtypeStruct((KP,), _i32),
            jax.ShapeDtypeStruct((NP,), _i32),
            jax.ShapeDtypeStruct((NP,), _i32),
            jax.ShapeDtypeStruct((KP,), _f32),
        ],
        mesh=_mesh(),
        scratch_types=[
            pltpu.VMEM((N + 16,), _f32),
            pltpu.VMEM((N + 16,), _i32),
            pltpu.VMEM((N + 16,), _i32),
            pltpu.VMEM((N + 16,), _i32),
            pltpu.VMEM((N + 16,), _i32),
            pltpu.VMEM((NB + 16,), _i32),
            pltpu.VMEM((N + 16,), _i32),
            pltpu.VMEM((NP,), _i32),
            pltpu.VMEM((NP,), _i32),
        ],
    )
    def k(score_hbm, perm_hbm, sel_hbm, nidx_hbm, ssort_hbm,
          sbuf, k0, p0, k1, p1, bins, dig, selb, nidxb):
        wid = _wid()

        @pl.when(wid == 0)
        def _():
            pltpu.sync_copy(score_hbm, sbuf.at[pl.ds(0, N)])

            def mk(j, _):
                v = sbuf[pl.ds(j * 16, 16)]
                b = lax.bitcast_convert_type(v, _i32)
                b = jnp.where(b == jnp.int32(-2147483648), 0, b)
                m = jnp.where(b < 0, ~b, b ^ jnp.int32(-2147483648))
                u = ~m
                k0[pl.ds(j * 16, 16)] = u
                p0[pl.ds(j * 16, 16)] = lax.iota(_i32, 16) + j * 16
                return 0

            lax.fori_loop(0, N // 16, mk, 0)

            for pno, shift in enumerate([0, 11, 22]):
                ks, ps = (k0, p0) if pno % 2 == 0 else (k1, p1)
                kd, pd = (k1, p1) if pno % 2 == 0 else (k0, p0)

                def zb(i, _):
                    bins[pl.ds(i * 16, 16)] = jnp.zeros((16,), _i32)
                    return 0

                lax.fori_loop(0, NB // 16, zb, 0)

                def dg(j, _, ks=ks, shift=shift):
                    u = ks[pl.ds(j * 16, 16)]
                    d = lax.shift_right_logical(u, shift) & (NB - 1)
                    dig[pl.ds(j * 16, 16)] = d
                    return 0

                lax.fori_loop(0, N // 16, dg, 0)

                def hist(i, _):
                    dv = dig[pl.ds(i * 16, 16)]
                    for l in range(16):
                        d = dv[l]
                        _sset(bins, d, _sget(bins, d) + 1)
                    return 0

                lax.fori_loop(0, N // 16, hist, 0)

                def scan(i, carry):
                    v = bins[pl.ds(i * 16, 16)]
                    inc = _prefix16(v)
                    exc = inc - v + carry
                    bins[pl.ds(i * 16, 16)] = exc
                    return carry + inc[15]

                lax.fori_loop(0, NB // 16, scan, jnp.int32(0))

                def perm_(i, _, ks=ks, ps=ps, kd=kd, pd=pd):
                    dv = dig[pl.ds(i * 16, 16)]
                    kv = ks[pl.ds(i * 16, 16)]
                    pv = ps[pl.ds(i * 16, 16)]
                    for l in range(16):
                        d = dv[l]
                        slot = _sget(bins, d)
                        _sset(bins, d, slot + 1)
                        _sset(kd, slot, kv[l])
                        _sset(pd, slot, pv[l])
                    return 0

                lax.fori_loop(0, N // 16, perm_, 0)

            pltpu.sync_copy(p1.at[pl.ds(0, KP)], perm_hbm)

            def inv(j, _):
                u = k1[pl.ds(j * 16, 16)]
                m = ~u
                b = jnp.where(m < 0, m ^ jnp.int32(-2147483648), ~m)
                sbuf[pl.ds(j * 16, 16)] = lax.bitcast_convert_type(b, _f32)
                return 0

            lax.fori_loop(0, KP // 16, inv, 0)
            pltpu.sync_copy(sbuf.at[pl.ds(0, KP)], ssort_hbm)

            def zsel(i, _):
                selb[pl.ds(i * 16, 16)] = jnp.zeros((16,), _i32)
                nidxb[pl.ds(i * 16, 16)] = jnp.zeros((16,), _i32)
                return 0

            lax.fori_loop(0, NP // 16, zsel, 0)

            def mark(r, _):
                pv = p1[pl.ds(r * 16, 16)]
                for l in range(16):
                    e = r * 16 + l

                    @pl.when(e < K)
                    def _():
                        i = pv[l]
                        _sset(selb, i, 1)
                        _sset(nidxb, i, e)
                return 0

            lax.fori_loop(0, (K + 15) // 16, mark, 0)
            pltpu.sync_copy(selb, sel_hbm)
            pltpu.sync_copy(nidxb, nidx_hbm)

    return k(score)


def _hp_deg2(h, perm, ssort, src, dst, sel, nidx):
    @functools.partial(
        pl.kernel,
        out_type=[
            jax.ShapeDtypeStruct((KP, D), _f32),
            jax.ShapeDtypeStruct((NW, 1, KO), _i32),
        ],
        mesh=_mesh(),
        scratch_types=[
            pltpu.VMEM((160,), _i32),
            pltpu.VMEM((160,), _f32),
            pltpu.VMEM((160, D), _f32),
            pltpu.VMEM((ECH,), _i32),
            pltpu.VMEM((ECH,), _i32),
            pltpu.VMEM((NP,), _i32),
            pltpu.VMEM((NP,), _i32),
            pltpu.VMEM((KO + 16,), _i32),
            pltpu.SemaphoreType.DMA,
        ],
    )
    def k(h_hbm, perm_hbm, ss_hbm, src_hbm, dst_hbm, sel_hbm, nidx_hbm,
          hp_hbm, d2_hbm, pbuf, scbuf, rows, dbuf, sbuf, selv, nidxv,
          hist, sem):
        wid = _wid()
        base = wid * 160
        pltpu.sync_copy(perm_hbm.at[pl.ds(base, 160)], pbuf)
        pltpu.sync_copy(ss_hbm.at[pl.ds(base, 160)], scbuf)
        pltpu.async_copy(h_hbm.at[pbuf], rows, sem).wait()

        def mul(g, _):
            sv = scbuf[pl.ds(g * 16, 16)]
            for l in range(16):
                i = g * 16 + l
                for c in range(8):
                    sl = pl.ds(c * 16, 16)
                    rows[i, sl] = rows[i, sl] * sv[l]
            return 0

        lax.fori_loop(0, 10, mul, 0)
        pltpu.sync_copy(rows, hp_hbm.at[pl.ds(base, 160)])

        pltpu.sync_copy(sel_hbm, selv)
        pltpu.sync_copy(nidx_hbm, nidxv)

        def zh(i, _):
            hist[pl.ds(i * 16, 16)] = jnp.zeros((16,), _i32)
            return 0

        lax.fori_loop(0, (KO + 16) // 16, zh, 0)
        pltpu.sync_copy(dst_hbm.at[pl.ds(wid * (E // NW), ECH)], dbuf)
        pltpu.sync_copy(src_hbm.at[pl.ds(wid * (E // NW), ECH)], sbuf)

        def upd(i, _):
            vs = sbuf[pl.ds(i * 16, 16)]
            vd = dbuf[pl.ds(i * 16, 16)]
            for l in range(16):
                s = vs[l]
                d = vd[l]
                ok = (_sget(selv, s) == 1) & (_sget(selv, d) == 1)

                @pl.when(ok)
                def _(d=d):
                    j = _sget(nidxv, d)
                    _sset(hist, j, _sget(hist, j) + 1)
            return 0

        lax.fori_loop(0, ECH // 16, upd, 0)
        pltpu.sync_copy(hist.at[pl.ds(0, KO)], d2_hbm.at[wid, 0])

    return k(h, perm, ssort, src, dst, sel, nidx)


def _t3(hp, W2, part2):
    def body(hp_ref, w_ref, p_ref, xw_ref, dis_ref):
        xw_ref[...] = jnp.dot(
            hp_ref[...], w_ref[...], preferred_element_type=_f32
        )
        deg = jnp.sum(p_ref[...], axis=0) + 1
        dis_ref[...] = lax.rsqrt(deg.astype(_f32))

    return pl.pallas_call(
        body,
        out_shape=[
            jax.ShapeDtypeStruct((KP, D), _f32),
            jax.ShapeDtypeStruct((1, KO), _f32),
        ],
    )(hp, W2, part2)


def _conv2(xw2, src, dst, sel, nidx, dis2, b2):
    @functools.partial(
        pl.kernel,
        out_type=jax.ShapeDtypeStruct((KO, D), _f32),
        mesh=_mesh(),
        scratch_types=[
            pltpu.VMEM((ECH,), _i32),
            pltpu.VMEM((ECH,), _i32),
            pltpu.VMEM((MCAP2 + 16,), _i32),
            pltpu.VMEM((MCAP2 + 16,), _i32),
            pltpu.VMEM((320, D), _f32),
            pltpu.VMEM((RB, D), _f32),
            pltpu.VMEM((NP,), _i32),
            pltpu.VMEM((NP,), _i32),
            pltpu.VMEM((KO + 16,), _f32),
            pltpu.VMEM((D,), _f32),
            pltpu.VMEM((16, D), _f32),
            pltpu.VMEM((16,), _i32),
            pltpu.SemaphoreType.DMA,
        ],
    )
    def k(xw_hbm, src_hbm, dst_hbm, sel_hbm, nidx_hbm, dis_hbm, b_hbm,
          out_hbm, dbuf, sbuf, msrc, mdst, acc, rows, selv, nidxv, disv,
          bv, selfb, jidx, sem):
        wid = _wid()
        n0, cnt = _wrange(wid)
        pltpu.sync_copy(sel_hbm, selv)
        pltpu.sync_copy(nidx_hbm, nidxv)
        pltpu.sync_copy(dis_hbm, disv.at[pl.ds(0, KO)])
        pltpu.sync_copy(b_hbm, bv)

        def zrow(i, _):
            for c in range(8):
                acc[i, pl.ds(c * 16, 16)] = jnp.zeros((16,), _f32)
            return 0

        lax.fori_loop(0, 320, zrow, 0)

        def zidx(i, _):
            msrc[pl.ds(i * 16, 16)] = jnp.zeros((16,), _i32)
            return 0

        lax.fori_loop(0, (MCAP2 + 16) // 16, zidx, 0)

        def chunk(ch, off):
            pltpu.sync_copy(dst_hbm.at[pl.ds(ch * ECH, ECH)], dbuf)
            pltpu.sync_copy(src_hbm.at[pl.ds(ch * ECH, ECH)], sbuf)

            def inner(j, off):
                vd = dbuf[pl.ds(j * 16, 16)]
                vs = sbuf[pl.ds(j * 16, 16)]
                m = (vd >= n0) & (vd < n0 + cnt)
                mv = jnp.where(m, 1, 0).astype(_i32)
                nm = mv[0]
                for l in range(1, 16):
                    nm = nm + mv[l]

                def wr(off):
                    for l in range(16):
                        s = vs[l]
                        d = vd[l]
                        ok = ((mv[l] == 1) & (_sget(selv, s) == 1)
                              & (_sget(selv, d) == 1))

                        @pl.when(ok)
                        def _(s=s, d=d, off=off):
                            _sset(mdst, off, d)
                            _sset(msrc, off, _sget(nidxv, s))
                        off = off + jnp.where(ok, 1, 0)
                    return off

                off = lax.cond(nm > 0, wr, lambda off: off, off)
                return jnp.minimum(off, MCAP2)

            return lax.fori_loop(0, ECH // 16, inner, off)

        off = lax.fori_loop(0, E // ECH, chunk, jnp.int32(0))

        nb = (off + RB - 1) // RB

        def blk(b, _):
            base = b * RB
            pltpu.async_copy(
                xw_hbm.at[msrc.at[pl.ds(base, RB)]], rows, sem
            ).wait()

            def egrp(g, _):
                gb = base + g * 16
                ns = msrc[pl.ds(gb, 16)]
                vd = mdst[pl.ds(gb, 16)]
                for l in range(16):
                    e = gb + l

                    @pl.when(e < off)
                    def _(l=l):
                        d = vd[l]
                        ln = d - n0
                        i = g * 16 + l
                        jj = _sget(nidxv, d)
                        nrm = _sget(disv, ns[l]) * _sget(disv, jj)
                        for c in range(8):
                            sl = pl.ds(c * 16, 16)
                            acc[ln, sl] = acc[ln, sl] + rows[i, sl] * nrm
                return 0

            lax.fori_loop(0, RB // 16, egrp, 0)
            return 0

        lax.fori_loop(0, nb, blk, 0)

        def grp(g, _):
            base_old = n0 + g * 16
            io16 = lax.iota(_i32, 16)
            jv = jnp.full((16,), KO - 1, _i32)
            okv = jnp.zeros((16,), _i32)
            for t in range(16):
                d = jnp.minimum(base_old + t, N - 1)
                okt = (g * 16 + t < cnt) & (_sget(selv, d) == 1)
                jt = jnp.where(okt, _sget(nidxv, d), KO - 1)
                jv = jnp.where(io16 == t, jt, jv)
                okv = jnp.where(io16 == t, jnp.where(okt, 1, 0), okv)
            jidx[...] = jv
            pltpu.async_copy(xw_hbm.at[jidx], selfb, sem).wait()

            for t in range(16):
                ln = g * 16 + t
                ok = okv[t] == 1

                @pl.when(ok)
                def _(t=t, ln=ln):
                    dj = _sget(disv, jv[t])
                    sn = dj * dj
                    for c in range(8):
                        sl = pl.ds(c * 16, 16)
                        v = acc[ln, sl] + selfb[t, sl] * sn
                        v = v + bv[sl]
                        selfb[t, sl] = jnp.where(v > 0.0, v, 0.0)

                @pl.when(jnp.logical_not(ok))
                def _(t=t):
                    for c in range(8):
                        selfb[t, pl.ds(c * 16, 16)] = jnp.zeros((16,), _f32)
            pltpu.async_copy(selfb, out_hbm.at[jidx], sem).wait()
            return 0

        lax.fori_loop(0, 20, grp, 0)

    return k(xw2, src, dst, sel, nidx, dis2, b2)


def kernel(x, edge_index, batch, W1, b1, w_rel, b_rel, w_root, W2, b2):
    del batch
    src = edge_index[0]
    dst = edge_index[1]
    part = _deg_partials(dst)
    xw, dis2d = _t1(x, W1, part)
    dis = dis2d.reshape(NP)
    h, e_src, e_dst, e_off = _seg_sum("conv1", xw, src, dst, dis=dis, bias=b1)
    aggr = _seg_sum("aggr", h, e_src, e_dst, offs=e_off)
    score = _t2(aggr, h, w_rel, b_rel, w_root).reshape(N)
    perm, sel, nidx, ssort = _topk(score)
    hp, part2 = _hp_deg2(h, perm, ssort, src, dst, sel, nidx)
    xw2, dis2_2d = _t3(hp, W2, part2)
    dis2 = dis2_2d.reshape(KO)
    out = _conv2(xw2, src, dst, sel, nidx, dis2, b2)
    return out[:K]
